# baseline (speedup 1.0000x reference)
import dataclasses
import functools
import jax
import jax.numpy as jnp
from jax import lax
from jax.experimental import pallas as pl
from jax.experimental.pallas import tpu as pltpu
from jax.experimental.pallas import tpu_sc as plsc

_B, _N, _H, _DIN, _E = 4, 2500, 128, 256, 320000
_NODES = _B * _N
_NPAD = 10240
_NC, _NS = 2, 16
_NW = _NC * _NS
_CHUNK = 128
_ITERS = 80
_EPAD = _NW * _ITERS * _CHUNK
_RPS = _NPAD // _NS
_OPS = _NODES // _NS

_mesh = plsc.VectorSubcoreMesh(core_axis_name="c", subcore_axis_name="s")

_cp = pltpu.CompilerParams()
if "needs_layout_passes" in pltpu.CompilerParams.__dataclass_fields__:
  _cp = dataclasses.replace(_cp, needs_layout_passes=False)

_cp_lin = pltpu.CompilerParams()
if "use_tc_tiling_on_sc" in pltpu.CompilerParams.__dataclass_fields__:
  _cp_lin = dataclasses.replace(_cp_lin, use_tc_tiling_on_sc=False)


@functools.partial(
    pl.kernel,
    out_type=[jax.ShapeDtypeStruct((_NW, _NPAD), jnp.float32),
              jax.ShapeDtypeStruct((_NW, _NPAD), jnp.float32)],
    mesh=_mesh,
    compiler_params=_cp,
    scratch_types=[pltpu.VMEM((_ITERS, _CHUNK), jnp.int32),
                   pltpu.VMEM((_ITERS, _CHUNK), jnp.int32),
                   pltpu.VMEM((_NPAD,), jnp.float32),
                   pltpu.VMEM((_NPAD,), jnp.float32)],
)
def _sc_degrees(src_h, dst_h, od_h, id_h, src_v, dst_v, od_v, id_v):
  cid = lax.axis_index("c")
  sid = lax.axis_index("s")
  wid = sid * _NC + cid

  @pl.loop(0, _NPAD, step=16)
  def _(i):
    od_v[pl.ds(i, 16)] = jnp.zeros((16,), jnp.float32)
    id_v[pl.ds(i, 16)] = jnp.zeros((16,), jnp.float32)

  pltpu.sync_copy(src_h.at[wid], src_v)
  pltpu.sync_copy(dst_h.at[wid], dst_v)

  ones = jnp.ones((16,), jnp.float32)

  @pl.loop(0, _ITERS)
  def _(i):
    @pl.loop(0, _CHUNK, step=16)
    def _(j):
      plsc.addupdate_scatter(od_v, [src_v[i, pl.ds(j, 16)]], ones)
      plsc.addupdate_scatter(id_v, [dst_v[i, pl.ds(j, 16)]], ones)

  pltpu.sync_copy(od_v, od_h.at[wid])
  pltpu.sync_copy(id_v, id_h.at[wid])


_HQ = _H // 4
_ITERS2 = _EPAD // (_NS * _CHUNK)


@functools.partial(
    pl.kernel,
    out_type=jax.ShapeDtypeStruct((_NC, 2, _NPAD, _HQ), jnp.float32),
    mesh=_mesh,
    compiler_params=_cp_lin,
    scratch_types=[pltpu.VMEM((_ITERS2, _CHUNK), jnp.int32),
                   pltpu.VMEM((_ITERS2, _CHUNK), jnp.int32),
                   pltpu.VMEM((_CHUNK, _HQ), jnp.float32),
                   pltpu.VMEM((_CHUNK, _HQ), jnp.float32),
                   pltpu.VMEM_SHARED((_NPAD, _HQ), jnp.float32),
                   pltpu.VMEM_SHARED((_NPAD, _HQ), jnp.float32)],
)
def _sc_aggregate(tab_h, src_h, dst_h, out_h, src_v, dst_v, rows_v,
                  zb_v, table_sh, agg_sh):
  cid = lax.axis_index("c")
  sid = lax.axis_index("s")

  @pl.loop(0, _CHUNK)
  def _(i):
    @pl.loop(0, _HQ, step=16)
    def _(j):
      zb_v[i, pl.ds(j, 16)] = jnp.zeros((16,), jnp.float32)

  pltpu.sync_copy(src_h.at[sid], src_v)
  pltpu.sync_copy(dst_h.at[sid], dst_v)

  base = sid * _RPS
  for q in (0, 1):
    @pl.loop(0, _RPS, step=_CHUNK)
    def _(r):
      pltpu.sync_copy(zb_v, agg_sh.at[pl.ds(base + r, _CHUNK)])

    pltpu.sync_copy(tab_h.at[cid * 2 + q, pl.ds(base, _RPS)],
                    table_sh.at[pl.ds(base, _RPS)])
    plsc.subcore_barrier()

    @pl.loop(0, _ITERS2)
    def _(i):
      pltpu.sync_copy(table_sh.at[src_v.at[i]], rows_v)
      pltpu.sync_copy(rows_v, agg_sh.at[dst_v.at[i]], add=True)

    plsc.subcore_barrier()
    pltpu.sync_copy(agg_sh.at[pl.ds(base, _RPS)],
                    out_h.at[cid, q, pl.ds(base, _RPS)])


_NR = _NPAD // 128


def _tc_norm_body(od_ref, id_ref, h_ref, tq_ref, normd_ref):
  od = jnp.sum(od_ref[...], axis=0)
  idg = jnp.sum(id_ref[...], axis=0)
  norm_src = jnp.where(od > 0, lax.rsqrt(jnp.maximum(od, 1.0)), 0.0)
  norm_dst = jnp.where(idg > 0, lax.rsqrt(jnp.maximum(idg, 1.0)), 0.0)
  table = h_ref[...] * norm_src[:, :, None]
  for q in range(4):
    tq_ref[q] = table[:, :, q * _HQ:(q + 1) * _HQ]
  normd_ref[...] = norm_dst


def _tc_norm(od_p, id_p, h3):
  return pl.pallas_call(
      _tc_norm_body,
      out_shape=[jax.ShapeDtypeStruct((4, _NR, 128, _HQ), jnp.float32),
                 jax.ShapeDtypeStruct((_NR, 128), jnp.float32)],
  )(od_p, id_p, h3)


def _tc_dense_body(parts_ref, normd_ref, wg_ref, bg_ref, x_ref, w4t_ref,
                   b4_ref, cprev_ref, h_ref, c_ref):
  agg = jnp.concatenate(
      [parts_ref[0, 0, 0], parts_ref[0, 1, 0],
       parts_ref[1, 0, 0], parts_ref[1, 1, 0]], axis=1)
  aggn = agg * normd_ref[0]
  hconv = jnp.dot(aggn, wg_ref[...],
                  preferred_element_type=jnp.float32) + bg_ref[...]
  xg = jnp.dot(x_ref[0], w4t_ref[...],
               preferred_element_type=jnp.float32) + b4_ref[...]
  i_t = jax.nn.sigmoid(hconv + xg[:, 0:_H])
  f_t = jax.nn.sigmoid(hconv + xg[:, _H:2 * _H])
  o_t = jax.nn.sigmoid(hconv + xg[:, 2 * _H:3 * _H])
  c_til = jnp.tanh(hconv + xg[:, 3 * _H:4 * _H])
  c_t = f_t * cprev_ref[0] + i_t * c_til
  h_ref[0] = o_t * jnp.tanh(c_t)
  c_ref[0] = c_t


def _tc_dense(parts_b, normd_col, w_g, b_g, x3, w4t, b4, c_prev):
  return pl.pallas_call(
      _tc_dense_body,
      grid=(_B,),
      in_specs=[
          pl.BlockSpec((_NC, 2, 1, _N, _HQ), lambda b: (0, 0, b, 0, 0)),
          pl.BlockSpec((1, _N, 1), lambda b: (b, 0, 0)),
          pl.BlockSpec((_H, _H), lambda b: (0, 0)),
          pl.BlockSpec((1, _H), lambda b: (0, 0)),
          pl.BlockSpec((1, 1, _DIN), lambda b: (b, 0, 0)),
          pl.BlockSpec((_DIN, 4 * _H), lambda b: (0, 0)),
          pl.BlockSpec((1, 4 * _H), lambda b: (0, 0)),
          pl.BlockSpec((1, _N, _H), lambda b: (b, 0, 0)),
      ],
      out_specs=[
          pl.BlockSpec((1, _N, _H), lambda b: (b, 0, 0)),
          pl.BlockSpec((1, _N, _H), lambda b: (b, 0, 0)),
      ],
      out_shape=[jax.ShapeDtypeStruct((_B, _N, _H), jnp.float32),
                 jax.ShapeDtypeStruct((_B, _N, _H), jnp.float32)],
  )(parts_b, normd_col, w_g, b_g, x3, w4t, b4, c_prev)


@jax.jit
def kernel(g_batch_edge_index, x, h_prev, c_prev,
           w_i, b_i, w_f, b_f, w_o, b_o, w_c, b_c, w_g, b_g):
  src = g_batch_edge_index[0].astype(jnp.int32)
  dst = g_batch_edge_index[1].astype(jnp.int32)
  pad = jnp.full((_EPAD - _E,), _NODES, jnp.int32)
  src_flat = jnp.concatenate([src, pad])
  src_p = src_flat.reshape(_NW, _ITERS, _CHUNK)
  src_p16 = src_flat.reshape(_NS, _ITERS2, _CHUNK)
  dst_flat = jnp.concatenate([dst, pad])
  dst_p = dst_flat.reshape(_NW, _ITERS, _CHUNK)
  dst_p16 = dst_flat.reshape(_NS, _ITERS2, _CHUNK)

  h_flat = h_prev.reshape(_NODES, _H)
  h_pad = jnp.concatenate(
      [h_flat, jnp.zeros((_NPAD - _NODES, _H), jnp.float32)])

  w4t = jnp.concatenate([w_i, w_f, w_o, w_c]).T
  b4 = jnp.concatenate([b_i, b_f, b_o, b_c])[None, :]

  od_p, id_p = _sc_degrees(src_p, dst_p)
  tq4, normd3 = _tc_norm(od_p.reshape(_NW, _NR, 128),
                         id_p.reshape(_NW, _NR, 128),
                         h_pad.reshape(_NR, 128, _H))
  tab = tq4.reshape(4, _NPAD, _HQ)
  normd_col = normd3.reshape(_NPAD)[:_NODES].reshape(_B, _N, 1)
  parts = _sc_aggregate(tab, src_p16, dst_p16)
  parts_b = parts[:, :, :_NODES].reshape(_NC, 2, _B, _N, _HQ)
  h_t, c_t = _tc_dense(parts_b, normd_col, w_g, b_g[None, :], x[:, None, :],
                       w4t, b4, c_prev)
  return (h_t, c_t)


# speedup vs baseline: 1.0804x; 1.0804x over previous
import dataclasses
import functools
import jax
import jax.numpy as jnp
from jax import lax
from jax.experimental import pallas as pl
from jax.experimental.pallas import tpu as pltpu
from jax.experimental.pallas import tpu_sc as plsc

_B, _N, _H, _DIN, _E = 4, 2500, 128, 256, 320000
_NODES = _B * _N
_NPAD = 10240
_NC, _NS = 2, 16
_NW = _NC * _NS
_CHUNK = 128
_ITERS = 80
_EPAD = _NW * _ITERS * _CHUNK
_RPS = _NPAD // _NS
_OPS = _NODES // _NS

_mesh = plsc.VectorSubcoreMesh(core_axis_name="c", subcore_axis_name="s")

_cp = pltpu.CompilerParams()
if "needs_layout_passes" in pltpu.CompilerParams.__dataclass_fields__:
  _cp = dataclasses.replace(_cp, needs_layout_passes=False)

_cp_lin = pltpu.CompilerParams()
if "use_tc_tiling_on_sc" in pltpu.CompilerParams.__dataclass_fields__:
  _cp_lin = dataclasses.replace(_cp_lin, use_tc_tiling_on_sc=False)


@functools.partial(
    pl.kernel,
    out_type=[jax.ShapeDtypeStruct((_NW, _NPAD), jnp.float32),
              jax.ShapeDtypeStruct((_NW, _NPAD), jnp.float32)],
    mesh=_mesh,
    compiler_params=_cp,
    scratch_types=[pltpu.VMEM((_ITERS, _CHUNK), jnp.int32),
                   pltpu.VMEM((_ITERS, _CHUNK), jnp.int32),
                   pltpu.VMEM((_NPAD,), jnp.float32),
                   pltpu.VMEM((_NPAD,), jnp.float32)],
)
def _sc_degrees(src_h, dst_h, od_h, id_h, src_v, dst_v, od_v, id_v):
  cid = lax.axis_index("c")
  sid = lax.axis_index("s")
  wid = sid * _NC + cid

  @pl.loop(0, _NPAD, step=16)
  def _(i):
    od_v[pl.ds(i, 16)] = jnp.zeros((16,), jnp.float32)
    id_v[pl.ds(i, 16)] = jnp.zeros((16,), jnp.float32)

  pltpu.sync_copy(src_h.at[wid], src_v)
  pltpu.sync_copy(dst_h.at[wid], dst_v)

  ones = jnp.ones((16,), jnp.float32)

  @pl.loop(0, _ITERS)
  def _(i):
    @pl.loop(0, _CHUNK, step=16)
    def _(j):
      plsc.addupdate_scatter(od_v, [src_v[i, pl.ds(j, 16)]], ones)
      plsc.addupdate_scatter(id_v, [dst_v[i, pl.ds(j, 16)]], ones)

  pltpu.sync_copy(od_v, od_h.at[wid])
  pltpu.sync_copy(id_v, id_h.at[wid])


_HQ = _H // 4
_ITERS2 = _EPAD // (_NS * _CHUNK)
_TROWS = _NODES + 16
_TPS = _TROWS // _NS


@functools.partial(
    pl.kernel,
    out_type=jax.ShapeDtypeStruct((_NC, 2, _NODES, _HQ), jnp.float32),
    mesh=_mesh,
    compiler_params=_cp_lin,
    scratch_types=[pltpu.VMEM((_ITERS2, _CHUNK), jnp.int32),
                   pltpu.VMEM((_ITERS2, _CHUNK), jnp.int32),
                   pltpu.VMEM((_CHUNK, _HQ), jnp.float32),
                   pltpu.VMEM((_CHUNK, _HQ), jnp.float32),
                   pltpu.VMEM_SHARED((_TROWS, _HQ), jnp.float32),
                   pltpu.VMEM_SHARED((_NODES, _HQ), jnp.float32)],
)
def _sc_aggregate(tab_h, src_h, dst_h, out_h, src_v, dst_v, rows_v,
                  zb_v, table_sh, agg_sh):
  cid = lax.axis_index("c")
  sid = lax.axis_index("s")

  @pl.loop(0, _CHUNK)
  def _(i):
    @pl.loop(0, _HQ, step=16)
    def _(j):
      zb_v[i, pl.ds(j, 16)] = jnp.zeros((16,), jnp.float32)

  pltpu.sync_copy(src_h.at[sid], src_v)
  pltpu.sync_copy(dst_h.at[sid], dst_v)

  obase = sid * _OPS
  tbase = sid * _TPS
  for q in (0, 1):
    @pl.loop(0, _OPS, step=125)
    def _(r):
      pltpu.sync_copy(zb_v.at[pl.ds(0, 125)], agg_sh.at[pl.ds(obase + r, 125)])

    pltpu.sync_copy(tab_h.at[cid * 2 + q, pl.ds(tbase, _TPS)],
                    table_sh.at[pl.ds(tbase, _TPS)])
    plsc.subcore_barrier()

    @pl.loop(0, _ITERS2)
    def _(i):
      pltpu.sync_copy(table_sh.at[src_v.at[i]], rows_v)
      pltpu.sync_copy(rows_v, agg_sh.at[dst_v.at[i]], add=True)

    plsc.subcore_barrier()
    pltpu.sync_copy(agg_sh.at[pl.ds(obase, _OPS)],
                    out_h.at[cid, q, pl.ds(obase, _OPS)])


_NR = _NPAD // 128


def _tc_norm_body(od_ref, id_ref, h_ref, tq_ref, normd_ref):
  od = jnp.sum(od_ref[...], axis=0)
  idg = jnp.sum(id_ref[...], axis=0)
  norm_src = jnp.where(od > 0, lax.rsqrt(jnp.maximum(od, 1.0)), 0.0)
  norm_dst = jnp.where(idg > 0, lax.rsqrt(jnp.maximum(idg, 1.0)), 0.0)
  table = h_ref[...] * norm_src[:, :, None]
  for q in range(4):
    tq_ref[q] = table[:, :, q * _HQ:(q + 1) * _HQ]
  normd_ref[...] = norm_dst


def _tc_norm(od_p, id_p, h3):
  return pl.pallas_call(
      _tc_norm_body,
      out_shape=[jax.ShapeDtypeStruct((4, _NR, 128, _HQ), jnp.float32),
                 jax.ShapeDtypeStruct((_NR, 128), jnp.float32)],
  )(od_p, id_p, h3)


def _tc_dense_body(parts_ref, normd_ref, wg_ref, bg_ref, x_ref, w4t_ref,
                   b4_ref, cprev_ref, h_ref, c_ref):
  agg = jnp.concatenate(
      [parts_ref[0, 0, 0], parts_ref[0, 1, 0],
       parts_ref[1, 0, 0], parts_ref[1, 1, 0]], axis=1)
  aggn = agg * normd_ref[0]
  hconv = jnp.dot(aggn, wg_ref[...],
                  preferred_element_type=jnp.float32) + bg_ref[...]
  xg = jnp.dot(x_ref[0], w4t_ref[...],
               preferred_element_type=jnp.float32) + b4_ref[...]
  i_t = jax.nn.sigmoid(hconv + xg[:, 0:_H])
  f_t = jax.nn.sigmoid(hconv + xg[:, _H:2 * _H])
  o_t = jax.nn.sigmoid(hconv + xg[:, 2 * _H:3 * _H])
  c_til = jnp.tanh(hconv + xg[:, 3 * _H:4 * _H])
  c_t = f_t * cprev_ref[0] + i_t * c_til
  h_ref[0] = o_t * jnp.tanh(c_t)
  c_ref[0] = c_t


def _tc_dense(parts_b, normd_col, w_g, b_g, x3, w4t, b4, c_prev):
  return pl.pallas_call(
      _tc_dense_body,
      grid=(_B,),
      in_specs=[
          pl.BlockSpec((_NC, 2, 1, _N, _HQ), lambda b: (0, 0, b, 0, 0)),
          pl.BlockSpec((1, _N, 1), lambda b: (b, 0, 0)),
          pl.BlockSpec((_H, _H), lambda b: (0, 0)),
          pl.BlockSpec((1, _H), lambda b: (0, 0)),
          pl.BlockSpec((1, 1, _DIN), lambda b: (b, 0, 0)),
          pl.BlockSpec((_DIN, 4 * _H), lambda b: (0, 0)),
          pl.BlockSpec((1, 4 * _H), lambda b: (0, 0)),
          pl.BlockSpec((1, _N, _H), lambda b: (b, 0, 0)),
      ],
      out_specs=[
          pl.BlockSpec((1, _N, _H), lambda b: (b, 0, 0)),
          pl.BlockSpec((1, _N, _H), lambda b: (b, 0, 0)),
      ],
      out_shape=[jax.ShapeDtypeStruct((_B, _N, _H), jnp.float32),
                 jax.ShapeDtypeStruct((_B, _N, _H), jnp.float32)],
  )(parts_b, normd_col, w_g, b_g, x3, w4t, b4, c_prev)


@jax.jit
def kernel(g_batch_edge_index, x, h_prev, c_prev,
           w_i, b_i, w_f, b_f, w_o, b_o, w_c, b_c, w_g, b_g):
  src = g_batch_edge_index[0].astype(jnp.int32)
  dst = g_batch_edge_index[1].astype(jnp.int32)
  pad = jnp.full((_EPAD - _E,), _NODES, jnp.int32)
  src_flat = jnp.concatenate([src, pad])
  src_p = src_flat.reshape(_NW, _ITERS, _CHUNK)
  src_p16 = src_flat.reshape(_NS, _ITERS2, _CHUNK)
  dst_p = jnp.concatenate([dst, pad]).reshape(_NW, _ITERS, _CHUNK)
  dst_p16 = jnp.concatenate(
      [dst, jnp.zeros((_EPAD - _E,), jnp.int32)]).reshape(_NS, _ITERS2, _CHUNK)

  h_flat = h_prev.reshape(_NODES, _H)
  h_pad = jnp.concatenate(
      [h_flat, jnp.zeros((_NPAD - _NODES, _H), jnp.float32)])

  w4t = jnp.concatenate([w_i, w_f, w_o, w_c]).T
  b4 = jnp.concatenate([b_i, b_f, b_o, b_c])[None, :]

  od_p, id_p = _sc_degrees(src_p, dst_p)
  tq4, normd3 = _tc_norm(od_p.reshape(_NW, _NR, 128),
                         id_p.reshape(_NW, _NR, 128),
                         h_pad.reshape(_NR, 128, _H))
  tab = tq4.reshape(4, _NPAD, _HQ)
  normd_col = normd3.reshape(_NPAD)[:_NODES].reshape(_B, _N, 1)
  parts = _sc_aggregate(tab, src_p16, dst_p16)
  parts_b = parts.reshape(_NC, 2, _B, _N, _HQ)
  h_t, c_t = _tc_dense(parts_b, normd_col, w_g, b_g[None, :], x[:, None, :],
                       w4t, b4, c_prev)
  return (h_t, c_t)


# speedup vs baseline: 1.2555x; 1.1622x over previous
import dataclasses
import functools
import jax
import jax.numpy as jnp
from jax import lax
from jax.experimental import pallas as pl
from jax.experimental.pallas import tpu as pltpu
from jax.experimental.pallas import tpu_sc as plsc

_B, _N, _H, _DIN, _E = 4, 2500, 128, 256, 320000
_NODES = _B * _N
_NPAD = 10240
_NC, _NS = 2, 16
_NW = _NC * _NS
_CHUNK = 128
_ITERS = 80
_EPAD = _NW * _ITERS * _CHUNK
_RPS = _NPAD // _NS
_OPS = _NODES // _NS

_mesh = plsc.VectorSubcoreMesh(core_axis_name="c", subcore_axis_name="s")

_cp = pltpu.CompilerParams()
if "needs_layout_passes" in pltpu.CompilerParams.__dataclass_fields__:
  _cp = dataclasses.replace(_cp, needs_layout_passes=False)

_cp_lin = pltpu.CompilerParams()
if "use_tc_tiling_on_sc" in pltpu.CompilerParams.__dataclass_fields__:
  _cp_lin = dataclasses.replace(_cp_lin, use_tc_tiling_on_sc=False)


@functools.partial(
    pl.kernel,
    out_type=[jax.ShapeDtypeStruct((_NW, _NPAD), jnp.float32),
              jax.ShapeDtypeStruct((_NW, _NPAD), jnp.float32)],
    mesh=_mesh,
    compiler_params=_cp,
    scratch_types=[pltpu.VMEM((_ITERS, _CHUNK), jnp.int32),
                   pltpu.VMEM((_ITERS, _CHUNK), jnp.int32),
                   pltpu.VMEM((_NPAD,), jnp.float32),
                   pltpu.VMEM((_NPAD,), jnp.float32)],
)
def _sc_degrees(src_h, dst_h, od_h, id_h, src_v, dst_v, od_v, id_v):
  cid = lax.axis_index("c")
  sid = lax.axis_index("s")
  wid = sid * _NC + cid

  @pl.loop(0, _NPAD, step=16)
  def _(i):
    od_v[pl.ds(i, 16)] = jnp.zeros((16,), jnp.float32)
    id_v[pl.ds(i, 16)] = jnp.zeros((16,), jnp.float32)

  pltpu.sync_copy(src_h.at[wid], src_v)
  pltpu.sync_copy(dst_h.at[wid], dst_v)

  ones = jnp.ones((16,), jnp.float32)

  @pl.loop(0, _ITERS)
  def _(i):
    @pl.loop(0, _CHUNK, step=16)
    def _(j):
      plsc.addupdate_scatter(od_v, [src_v[i, pl.ds(j, 16)]], ones)
      plsc.addupdate_scatter(id_v, [dst_v[i, pl.ds(j, 16)]], ones)

  pltpu.sync_copy(od_v, od_h.at[wid])
  pltpu.sync_copy(id_v, id_h.at[wid])


_HQ = _H // 4
_ITERS2 = _EPAD // (_NS * _CHUNK)
_TROWS = _NODES + 16
_TPS = _TROWS // _NS


@functools.partial(
    pl.kernel,
    out_type=jax.ShapeDtypeStruct((_NC, 2, _NODES, _HQ), jnp.float32),
    mesh=_mesh,
    compiler_params=_cp_lin,
    scratch_types=[pltpu.VMEM((_ITERS2, _CHUNK), jnp.int32),
                   pltpu.VMEM((_ITERS2, _CHUNK), jnp.int32),
                   pltpu.VMEM((_CHUNK, _HQ), jnp.float32),
                   pltpu.VMEM((_CHUNK, _HQ), jnp.float32),
                   pltpu.VMEM((_CHUNK, _HQ), jnp.float32),
                   pltpu.VMEM_SHARED((_TROWS, _HQ), jnp.float32),
                   pltpu.VMEM_SHARED((_NODES, _HQ), jnp.float32),
                   pltpu.SemaphoreType.DMA,
                   pltpu.SemaphoreType.DMA],
)
def _sc_aggregate(tab_h, src_h, dst_h, out_h, src_v, dst_v, rows_v,
                  rows1_v, zb_v, table_sh, agg_sh, gsem0, gsem1):
  cid = lax.axis_index("c")
  sid = lax.axis_index("s")

  @pl.loop(0, _CHUNK)
  def _(i):
    @pl.loop(0, _HQ, step=16)
    def _(j):
      zb_v[i, pl.ds(j, 16)] = jnp.zeros((16,), jnp.float32)

  pltpu.sync_copy(src_h.at[sid], src_v)
  pltpu.sync_copy(dst_h.at[sid], dst_v)

  obase = sid * _OPS
  tbase = sid * _TPS
  for q in (0, 1):
    @pl.loop(0, _OPS, step=125)
    def _(r):
      pltpu.sync_copy(zb_v.at[pl.ds(0, 125)], agg_sh.at[pl.ds(obase + r, 125)])

    pltpu.sync_copy(tab_h.at[cid * 2 + q, pl.ds(tbase, _TPS)],
                    table_sh.at[pl.ds(tbase, _TPS)])
    plsc.subcore_barrier()

    pltpu.async_copy(table_sh.at[src_v.at[0]], rows_v, gsem0)

    @pl.loop(0, _ITERS2, step=2)
    def _(c):
      pltpu.make_async_copy(table_sh.at[src_v.at[c]], rows_v, gsem0).wait()
      pltpu.async_copy(table_sh.at[src_v.at[c + 1]], rows1_v, gsem1)
      pltpu.sync_copy(rows_v, agg_sh.at[dst_v.at[c]], add=True)

      pltpu.make_async_copy(table_sh.at[src_v.at[c + 1]], rows1_v,
                            gsem1).wait()
      @pl.when(c + 2 < _ITERS2)
      def _():
        pltpu.async_copy(table_sh.at[src_v.at[c + 2]], rows_v, gsem0)
      pltpu.sync_copy(rows1_v, agg_sh.at[dst_v.at[c + 1]], add=True)

    plsc.subcore_barrier()
    pltpu.sync_copy(agg_sh.at[pl.ds(obase, _OPS)],
                    out_h.at[cid, q, pl.ds(obase, _OPS)])


_NR = _NPAD // 128


def _tc_norm_body(od_ref, id_ref, h_ref, tq_ref, normd_ref):
  od = jnp.sum(od_ref[...], axis=0)
  idg = jnp.sum(id_ref[...], axis=0)
  norm_src = jnp.where(od > 0, lax.rsqrt(jnp.maximum(od, 1.0)), 0.0)
  norm_dst = jnp.where(idg > 0, lax.rsqrt(jnp.maximum(idg, 1.0)), 0.0)
  table = h_ref[...] * norm_src[:, :, None]
  for q in range(4):
    tq_ref[q] = table[:, :, q * _HQ:(q + 1) * _HQ]
  normd_ref[...] = norm_dst


def _tc_norm(od_p, id_p, h3):
  return pl.pallas_call(
      _tc_norm_body,
      out_shape=[jax.ShapeDtypeStruct((4, _NR, 128, _HQ), jnp.float32),
                 jax.ShapeDtypeStruct((_NR, 128), jnp.float32)],
  )(od_p, id_p, h3)


def _tc_dense_body(parts_ref, normd_ref, wg_ref, bg_ref, x_ref, w4t_ref,
                   b4_ref, cprev_ref, h_ref, c_ref):
  agg = jnp.concatenate(
      [parts_ref[0, 0, 0], parts_ref[0, 1, 0],
       parts_ref[1, 0, 0], parts_ref[1, 1, 0]], axis=1)
  aggn = agg * normd_ref[0]
  hconv = jnp.dot(aggn, wg_ref[...],
                  preferred_element_type=jnp.float32) + bg_ref[...]
  xg = jnp.dot(x_ref[0], w4t_ref[...],
               preferred_element_type=jnp.float32) + b4_ref[...]
  i_t = jax.nn.sigmoid(hconv + xg[:, 0:_H])
  f_t = jax.nn.sigmoid(hconv + xg[:, _H:2 * _H])
  o_t = jax.nn.sigmoid(hconv + xg[:, 2 * _H:3 * _H])
  c_til = jnp.tanh(hconv + xg[:, 3 * _H:4 * _H])
  c_t = f_t * cprev_ref[0] + i_t * c_til
  h_ref[0] = o_t * jnp.tanh(c_t)
  c_ref[0] = c_t


def _tc_dense(parts_b, normd_col, w_g, b_g, x3, w4t, b4, c_prev):
  return pl.pallas_call(
      _tc_dense_body,
      grid=(_B,),
      in_specs=[
          pl.BlockSpec((_NC, 2, 1, _N, _HQ), lambda b: (0, 0, b, 0, 0)),
          pl.BlockSpec((1, _N, 1), lambda b: (b, 0, 0)),
          pl.BlockSpec((_H, _H), lambda b: (0, 0)),
          pl.BlockSpec((1, _H), lambda b: (0, 0)),
          pl.BlockSpec((1, 1, _DIN), lambda b: (b, 0, 0)),
          pl.BlockSpec((_DIN, 4 * _H), lambda b: (0, 0)),
          pl.BlockSpec((1, 4 * _H), lambda b: (0, 0)),
          pl.BlockSpec((1, _N, _H), lambda b: (b, 0, 0)),
      ],
      out_specs=[
          pl.BlockSpec((1, _N, _H), lambda b: (b, 0, 0)),
          pl.BlockSpec((1, _N, _H), lambda b: (b, 0, 0)),
      ],
      out_shape=[jax.ShapeDtypeStruct((_B, _N, _H), jnp.float32),
                 jax.ShapeDtypeStruct((_B, _N, _H), jnp.float32)],
  )(parts_b, normd_col, w_g, b_g, x3, w4t, b4, c_prev)


@jax.jit
def kernel(g_batch_edge_index, x, h_prev, c_prev,
           w_i, b_i, w_f, b_f, w_o, b_o, w_c, b_c, w_g, b_g):
  src = g_batch_edge_index[0].astype(jnp.int32)
  dst = g_batch_edge_index[1].astype(jnp.int32)
  pad = jnp.full((_EPAD - _E,), _NODES, jnp.int32)
  src_flat = jnp.concatenate([src, pad])
  src_p = src_flat.reshape(_NW, _ITERS, _CHUNK)
  src_p16 = src_flat.reshape(_NS, _ITERS2, _CHUNK)
  dst_p = jnp.concatenate([dst, pad]).reshape(_NW, _ITERS, _CHUNK)
  dst_p16 = jnp.concatenate(
      [dst, jnp.zeros((_EPAD - _E,), jnp.int32)]).reshape(_NS, _ITERS2, _CHUNK)

  h_flat = h_prev.reshape(_NODES, _H)
  h_pad = jnp.concatenate(
      [h_flat, jnp.zeros((_NPAD - _NODES, _H), jnp.float32)])

  w4t = jnp.concatenate([w_i, w_f, w_o, w_c]).T
  b4 = jnp.concatenate([b_i, b_f, b_o, b_c])[None, :]

  od_p, id_p = _sc_degrees(src_p, dst_p)
  tq4, normd3 = _tc_norm(od_p.reshape(_NW, _NR, 128),
                         id_p.reshape(_NW, _NR, 128),
                         h_pad.reshape(_NR, 128, _H))
  tab = tq4.reshape(4, _NPAD, _HQ)
  normd_col = normd3.reshape(_NPAD)[:_NODES].reshape(_B, _N, 1)
  parts = _sc_aggregate(tab, src_p16, dst_p16)
  parts_b = parts.reshape(_NC, 2, _B, _N, _HQ)
  h_t, c_t = _tc_dense(parts_b, normd_col, w_g, b_g[None, :], x[:, None, :],
                       w4t, b4, c_prev)
  return (h_t, c_t)


# speedup vs baseline: 1.3034x; 1.0381x over previous
import dataclasses
import functools
import jax
import jax.numpy as jnp
from jax import lax
from jax.experimental import pallas as pl
from jax.experimental.pallas import tpu as pltpu
from jax.experimental.pallas import tpu_sc as plsc

_B, _N, _H, _DIN, _E = 4, 2500, 128, 256, 320000
_NODES = _B * _N
_NPAD = 10240
_NC, _NS = 2, 16
_NW = _NC * _NS
_CHUNK = 128
_ITERS = 80
_EPAD = _NW * _ITERS * _CHUNK
_RPS = _NPAD // _NS
_OPS = _NODES // _NS

_mesh = plsc.VectorSubcoreMesh(core_axis_name="c", subcore_axis_name="s")

_cp = pltpu.CompilerParams()
if "needs_layout_passes" in pltpu.CompilerParams.__dataclass_fields__:
  _cp = dataclasses.replace(_cp, needs_layout_passes=False)

_cp_lin = pltpu.CompilerParams()
if "use_tc_tiling_on_sc" in pltpu.CompilerParams.__dataclass_fields__:
  _cp_lin = dataclasses.replace(_cp_lin, use_tc_tiling_on_sc=False)


@functools.partial(
    pl.kernel,
    out_type=[jax.ShapeDtypeStruct((_NW, _NPAD), jnp.float32),
              jax.ShapeDtypeStruct((_NW, _NPAD), jnp.float32)],
    mesh=_mesh,
    compiler_params=_cp,
    scratch_types=[pltpu.VMEM((_ITERS, _CHUNK), jnp.int32),
                   pltpu.VMEM((_ITERS, _CHUNK), jnp.int32),
                   pltpu.VMEM((_NPAD,), jnp.float32),
                   pltpu.VMEM((_NPAD,), jnp.float32)],
)
def _sc_degrees(src_h, dst_h, od_h, id_h, src_v, dst_v, od_v, id_v):
  cid = lax.axis_index("c")
  sid = lax.axis_index("s")
  wid = sid * _NC + cid

  @pl.loop(0, _NPAD, step=16)
  def _(i):
    od_v[pl.ds(i, 16)] = jnp.zeros((16,), jnp.float32)
    id_v[pl.ds(i, 16)] = jnp.zeros((16,), jnp.float32)

  pltpu.sync_copy(src_h.at[wid], src_v)
  pltpu.sync_copy(dst_h.at[wid], dst_v)

  ones = jnp.ones((16,), jnp.float32)

  @pl.loop(0, _ITERS)
  def _(i):
    @pl.loop(0, _CHUNK, step=16)
    def _(j):
      plsc.addupdate_scatter(od_v, [src_v[i, pl.ds(j, 16)]], ones)
      plsc.addupdate_scatter(id_v, [dst_v[i, pl.ds(j, 16)]], ones)

  pltpu.sync_copy(od_v, od_h.at[wid])
  pltpu.sync_copy(id_v, id_h.at[wid])


_HQ = _H // 4
_ITERS2 = _EPAD // (_NS * _CHUNK)
_TROWS = _NODES + 16
_TPS = _TROWS // _NS


@functools.partial(
    pl.kernel,
    out_type=jax.ShapeDtypeStruct((_NC, 2, _NODES, _HQ), jnp.float32),
    mesh=_mesh,
    compiler_params=_cp_lin,
    scratch_types=[pltpu.VMEM((_ITERS2, _CHUNK), jnp.int32),
                   pltpu.VMEM((_ITERS2, _CHUNK), jnp.int32),
                   pltpu.VMEM((_CHUNK, _HQ), jnp.float32),
                   pltpu.VMEM((_CHUNK, _HQ), jnp.float32),
                   pltpu.VMEM((_CHUNK, _HQ), jnp.float32),
                   pltpu.VMEM((_CHUNK, _HQ), jnp.float32),
                   pltpu.VMEM((_CHUNK, _HQ), jnp.float32),
                   pltpu.VMEM_SHARED((_TROWS, _HQ), jnp.float32),
                   pltpu.VMEM_SHARED((_NODES, _HQ), jnp.float32),
                   pltpu.SemaphoreType.DMA,
                   pltpu.SemaphoreType.DMA,
                   pltpu.SemaphoreType.DMA,
                   pltpu.SemaphoreType.DMA],
)
def _sc_aggregate(tab_h, src_h, dst_h, out_h, src_v, dst_v, rows0_v,
                  rows1_v, rows2_v, rows3_v, zb_v, table_sh, agg_sh,
                  gsem0, gsem1, gsem2, gsem3):
  cid = lax.axis_index("c")
  sid = lax.axis_index("s")

  @pl.loop(0, _CHUNK)
  def _(i):
    @pl.loop(0, _HQ, step=16)
    def _(j):
      zb_v[i, pl.ds(j, 16)] = jnp.zeros((16,), jnp.float32)

  pltpu.sync_copy(src_h.at[sid], src_v)
  pltpu.sync_copy(dst_h.at[sid], dst_v)

  obase = sid * _OPS
  tbase = sid * _TPS
  for q in (0, 1):
    @pl.loop(0, _OPS, step=125)
    def _(r):
      pltpu.sync_copy(zb_v.at[pl.ds(0, 125)], agg_sh.at[pl.ds(obase + r, 125)])

    pltpu.sync_copy(tab_h.at[cid * 2 + q, pl.ds(tbase, _TPS)],
                    table_sh.at[pl.ds(tbase, _TPS)])
    plsc.subcore_barrier()

    bufs = (rows0_v, rows1_v, rows2_v, rows3_v)
    sems = (gsem0, gsem1, gsem2, gsem3)
    for b in range(4):
      pltpu.async_copy(table_sh.at[src_v.at[b]], bufs[b], sems[b])

    @pl.loop(0, _ITERS2, step=4)
    def _(c):
      for b in range(4):
        pltpu.make_async_copy(table_sh.at[src_v.at[c + b]], bufs[b],
                              sems[b]).wait()
        pltpu.sync_copy(bufs[b], agg_sh.at[dst_v.at[c + b]], add=True)

        @pl.when(c + b + 4 < _ITERS2)
        def _():
          pltpu.async_copy(table_sh.at[src_v.at[c + b + 4]], bufs[b],
                           sems[b])

    plsc.subcore_barrier()
    pltpu.sync_copy(agg_sh.at[pl.ds(obase, _OPS)],
                    out_h.at[cid, q, pl.ds(obase, _OPS)])


_NR = _NPAD // 128


def _tc_norm_body(od_ref, id_ref, h_ref, tq_ref, normd_ref):
  od = jnp.sum(od_ref[...], axis=0)
  idg = jnp.sum(id_ref[...], axis=0)
  norm_src = jnp.where(od > 0, lax.rsqrt(jnp.maximum(od, 1.0)), 0.0)
  norm_dst = jnp.where(idg > 0, lax.rsqrt(jnp.maximum(idg, 1.0)), 0.0)
  table = h_ref[...] * norm_src[:, :, None]
  for q in range(4):
    tq_ref[q] = table[:, :, q * _HQ:(q + 1) * _HQ]
  normd_ref[...] = norm_dst


def _tc_norm(od_p, id_p, h3):
  return pl.pallas_call(
      _tc_norm_body,
      out_shape=[jax.ShapeDtypeStruct((4, _NR, 128, _HQ), jnp.float32),
                 jax.ShapeDtypeStruct((_NR, 128), jnp.float32)],
  )(od_p, id_p, h3)


def _tc_dense_body(parts_ref, normd_ref, wg_ref, bg_ref, x_ref, w4t_ref,
                   b4_ref, cprev_ref, h_ref, c_ref):
  agg = jnp.concatenate(
      [parts_ref[0, 0, 0], parts_ref[0, 1, 0],
       parts_ref[1, 0, 0], parts_ref[1, 1, 0]], axis=1)
  aggn = agg * normd_ref[0]
  hconv = jnp.dot(aggn, wg_ref[...],
                  preferred_element_type=jnp.float32) + bg_ref[...]
  xg = jnp.dot(x_ref[0], w4t_ref[...],
               preferred_element_type=jnp.float32) + b4_ref[...]
  i_t = jax.nn.sigmoid(hconv + xg[:, 0:_H])
  f_t = jax.nn.sigmoid(hconv + xg[:, _H:2 * _H])
  o_t = jax.nn.sigmoid(hconv + xg[:, 2 * _H:3 * _H])
  c_til = jnp.tanh(hconv + xg[:, 3 * _H:4 * _H])
  c_t = f_t * cprev_ref[0] + i_t * c_til
  h_ref[0] = o_t * jnp.tanh(c_t)
  c_ref[0] = c_t


def _tc_dense(parts_b, normd_col, w_g, b_g, x3, w4t, b4, c_prev):
  return pl.pallas_call(
      _tc_dense_body,
      grid=(_B,),
      in_specs=[
          pl.BlockSpec((_NC, 2, 1, _N, _HQ), lambda b: (0, 0, b, 0, 0)),
          pl.BlockSpec((1, _N, 1), lambda b: (b, 0, 0)),
          pl.BlockSpec((_H, _H), lambda b: (0, 0)),
          pl.BlockSpec((1, _H), lambda b: (0, 0)),
          pl.BlockSpec((1, 1, _DIN), lambda b: (b, 0, 0)),
          pl.BlockSpec((_DIN, 4 * _H), lambda b: (0, 0)),
          pl.BlockSpec((1, 4 * _H), lambda b: (0, 0)),
          pl.BlockSpec((1, _N, _H), lambda b: (b, 0, 0)),
      ],
      out_specs=[
          pl.BlockSpec((1, _N, _H), lambda b: (b, 0, 0)),
          pl.BlockSpec((1, _N, _H), lambda b: (b, 0, 0)),
      ],
      out_shape=[jax.ShapeDtypeStruct((_B, _N, _H), jnp.float32),
                 jax.ShapeDtypeStruct((_B, _N, _H), jnp.float32)],
  )(parts_b, normd_col, w_g, b_g, x3, w4t, b4, c_prev)


@jax.jit
def kernel(g_batch_edge_index, x, h_prev, c_prev,
           w_i, b_i, w_f, b_f, w_o, b_o, w_c, b_c, w_g, b_g):
  src = g_batch_edge_index[0].astype(jnp.int32)
  dst = g_batch_edge_index[1].astype(jnp.int32)
  pad = jnp.full((_EPAD - _E,), _NODES, jnp.int32)
  src_flat = jnp.concatenate([src, pad])
  src_p = src_flat.reshape(_NW, _ITERS, _CHUNK)
  src_p16 = src_flat.reshape(_NS, _ITERS2, _CHUNK)
  dst_p = jnp.concatenate([dst, pad]).reshape(_NW, _ITERS, _CHUNK)
  dst_p16 = jnp.concatenate(
      [dst, jnp.zeros((_EPAD - _E,), jnp.int32)]).reshape(_NS, _ITERS2, _CHUNK)

  h_flat = h_prev.reshape(_NODES, _H)
  h_pad = jnp.concatenate(
      [h_flat, jnp.zeros((_NPAD - _NODES, _H), jnp.float32)])

  w4t = jnp.concatenate([w_i, w_f, w_o, w_c]).T
  b4 = jnp.concatenate([b_i, b_f, b_o, b_c])[None, :]

  od_p, id_p = _sc_degrees(src_p, dst_p)
  tq4, normd3 = _tc_norm(od_p.reshape(_NW, _NR, 128),
                         id_p.reshape(_NW, _NR, 128),
                         h_pad.reshape(_NR, 128, _H))
  tab = tq4.reshape(4, _NPAD, _HQ)
  normd_col = normd3.reshape(_NPAD)[:_NODES].reshape(_B, _N, 1)
  parts = _sc_aggregate(tab, src_p16, dst_p16)
  parts_b = parts.reshape(_NC, 2, _B, _N, _HQ)
  h_t, c_t = _tc_dense(parts_b, normd_col, w_g, b_g[None, :], x[:, None, :],
                       w4t, b4, c_prev)
  return (h_t, c_t)


# speedup vs baseline: 1.3777x; 1.0570x over previous
import dataclasses
import functools
import jax
import jax.numpy as jnp
from jax import lax
from jax.experimental import pallas as pl
from jax.experimental.pallas import tpu as pltpu
from jax.experimental.pallas import tpu_sc as plsc

_B, _N, _H, _DIN, _E = 4, 2500, 128, 256, 320000
_NODES = _B * _N
_NPAD = 10240
_NC, _NS = 2, 16
_NW = _NC * _NS
_CHUNK = 128
_ITERS = 80
_EPAD = _NW * _ITERS * _CHUNK
_RPS = _NPAD // _NS
_OPS = _NODES // _NS

_mesh = plsc.VectorSubcoreMesh(core_axis_name="c", subcore_axis_name="s")

_cp = pltpu.CompilerParams()
if "needs_layout_passes" in pltpu.CompilerParams.__dataclass_fields__:
  _cp = dataclasses.replace(_cp, needs_layout_passes=False)

_cp_lin = pltpu.CompilerParams()
if "use_tc_tiling_on_sc" in pltpu.CompilerParams.__dataclass_fields__:
  _cp_lin = dataclasses.replace(_cp_lin, use_tc_tiling_on_sc=False)


@functools.partial(
    pl.kernel,
    out_type=[jax.ShapeDtypeStruct((_NW, _NPAD), jnp.float32),
              jax.ShapeDtypeStruct((_NW, _NPAD), jnp.float32)],
    mesh=_mesh,
    compiler_params=_cp,
    scratch_types=[pltpu.VMEM((_ITERS, _CHUNK), jnp.int32),
                   pltpu.VMEM((_ITERS, _CHUNK), jnp.int32),
                   pltpu.VMEM((_NPAD,), jnp.float32),
                   pltpu.VMEM((_NPAD,), jnp.float32)],
)
def _sc_degrees(src_h, dst_h, od_h, id_h, src_v, dst_v, od_v, id_v):
  cid = lax.axis_index("c")
  sid = lax.axis_index("s")
  wid = sid * _NC + cid

  @pl.loop(0, _NPAD, step=16)
  def _(i):
    od_v[pl.ds(i, 16)] = jnp.zeros((16,), jnp.float32)
    id_v[pl.ds(i, 16)] = jnp.zeros((16,), jnp.float32)

  pltpu.sync_copy(src_h.at[wid], src_v)
  pltpu.sync_copy(dst_h.at[wid], dst_v)

  ones = jnp.ones((16,), jnp.float32)

  @pl.loop(0, _ITERS)
  def _(i):
    @pl.loop(0, _CHUNK, step=16)
    def _(j):
      plsc.addupdate_scatter(od_v, [src_v[i, pl.ds(j, 16)]], ones)
      plsc.addupdate_scatter(id_v, [dst_v[i, pl.ds(j, 16)]], ones)

  pltpu.sync_copy(od_v, od_h.at[wid])
  pltpu.sync_copy(id_v, id_h.at[wid])


_HQ = _H // 4
_ITERS2 = _EPAD // (_NS * _CHUNK)
_TROWS = _NODES + 16
_TPS = _TROWS // _NS


@functools.partial(
    pl.kernel,
    out_type=jax.ShapeDtypeStruct((_NC, 2, _NODES, _HQ), jnp.float32),
    mesh=_mesh,
    compiler_params=_cp_lin,
    scratch_types=([pltpu.VMEM((_ITERS2, _CHUNK), jnp.int32),
                    pltpu.VMEM((_ITERS2, _CHUNK), jnp.int32)] +
                   [pltpu.VMEM((_CHUNK, _HQ), jnp.float32)] * 9 +
                   [pltpu.VMEM_SHARED((_TROWS, _HQ), jnp.float32),
                    pltpu.VMEM_SHARED((_NODES, _HQ), jnp.float32)] +
                   [pltpu.SemaphoreType.DMA] * 16),
)
def _sc_aggregate(tab_h, src_h, dst_h, out_h, src_v, dst_v,
                  r0, r1, r2, r3, r4, r5, r6, r7, zb_v, table_sh, agg_sh,
                  g0, g1, g2, g3, g4, g5, g6, g7,
                  s0, s1, s2, s3, s4, s5, s6, s7):
  cid = lax.axis_index("c")
  sid = lax.axis_index("s")

  @pl.loop(0, _CHUNK)
  def _(i):
    @pl.loop(0, _HQ, step=16)
    def _(j):
      zb_v[i, pl.ds(j, 16)] = jnp.zeros((16,), jnp.float32)

  pltpu.sync_copy(src_h.at[sid], src_v)
  pltpu.sync_copy(dst_h.at[sid], dst_v)

  obase = sid * _OPS
  tbase = sid * _TPS
  for q in (0, 1):
    @pl.loop(0, _OPS, step=125)
    def _(r):
      pltpu.sync_copy(zb_v.at[pl.ds(0, 125)], agg_sh.at[pl.ds(obase + r, 125)])

    pltpu.sync_copy(tab_h.at[cid * 2 + q, pl.ds(tbase, _TPS)],
                    table_sh.at[pl.ds(tbase, _TPS)])
    plsc.subcore_barrier()

    bufs = (r0, r1, r2, r3, r4, r5, r6, r7)
    gs = (g0, g1, g2, g3, g4, g5, g6, g7)
    ss = (s0, s1, s2, s3, s4, s5, s6, s7)
    for b in range(8):
      pltpu.async_copy(table_sh.at[src_v.at[b]], bufs[b], gs[b])

    @pl.loop(0, _ITERS2, step=8)
    def _(c):
      for b in range(8):
        cc = c + b
        pltpu.make_async_copy(table_sh.at[src_v.at[cc]], bufs[b],
                              gs[b]).wait()
        pltpu.async_copy(bufs[b], agg_sh.at[dst_v.at[cc]], ss[b], add=True)
        b4 = (b + 4) % 8

        @pl.when((cc >= 4) & (cc + 4 < _ITERS2))
        def _():
          pltpu.make_async_copy(bufs[b4], agg_sh.at[dst_v.at[cc - 4]],
                                ss[b4]).wait()
          pltpu.async_copy(table_sh.at[src_v.at[cc + 4]], bufs[b4], gs[b4])

    for b in range(8):
      pltpu.make_async_copy(bufs[b], agg_sh.at[dst_v.at[_ITERS2 - 8 + b]],
                            ss[b]).wait()

    plsc.subcore_barrier()
    pltpu.sync_copy(agg_sh.at[pl.ds(obase, _OPS)],
                    out_h.at[cid, q, pl.ds(obase, _OPS)])


_NR = _NPAD // 128


def _tc_norm_body(od_ref, id_ref, h_ref, tq_ref, normd_ref):
  od = jnp.sum(od_ref[...], axis=0)
  idg = jnp.sum(id_ref[...], axis=0)
  norm_src = jnp.where(od > 0, lax.rsqrt(jnp.maximum(od, 1.0)), 0.0)
  norm_dst = jnp.where(idg > 0, lax.rsqrt(jnp.maximum(idg, 1.0)), 0.0)
  table = h_ref[...] * norm_src[:, :, None]
  for q in range(4):
    tq_ref[q] = table[:, :, q * _HQ:(q + 1) * _HQ]
  normd_ref[...] = norm_dst


def _tc_norm(od_p, id_p, h3):
  return pl.pallas_call(
      _tc_norm_body,
      out_shape=[jax.ShapeDtypeStruct((4, _NR, 128, _HQ), jnp.float32),
                 jax.ShapeDtypeStruct((_NR, 128), jnp.float32)],
  )(od_p, id_p, h3)


def _tc_dense_body(parts_ref, normd_ref, wg_ref, bg_ref, x_ref, w4t_ref,
                   b4_ref, cprev_ref, h_ref, c_ref):
  agg = jnp.concatenate(
      [parts_ref[0, 0, 0], parts_ref[0, 1, 0],
       parts_ref[1, 0, 0], parts_ref[1, 1, 0]], axis=1)
  aggn = agg * normd_ref[0]
  hconv = jnp.dot(aggn, wg_ref[...],
                  preferred_element_type=jnp.float32) + bg_ref[...]
  xg = jnp.dot(x_ref[0], w4t_ref[...],
               preferred_element_type=jnp.float32) + b4_ref[...]
  i_t = jax.nn.sigmoid(hconv + xg[:, 0:_H])
  f_t = jax.nn.sigmoid(hconv + xg[:, _H:2 * _H])
  o_t = jax.nn.sigmoid(hconv + xg[:, 2 * _H:3 * _H])
  c_til = jnp.tanh(hconv + xg[:, 3 * _H:4 * _H])
  c_t = f_t * cprev_ref[0] + i_t * c_til
  h_ref[0] = o_t * jnp.tanh(c_t)
  c_ref[0] = c_t


def _tc_dense(parts_b, normd_col, w_g, b_g, x3, w4t, b4, c_prev):
  return pl.pallas_call(
      _tc_dense_body,
      grid=(_B,),
      in_specs=[
          pl.BlockSpec((_NC, 2, 1, _N, _HQ), lambda b: (0, 0, b, 0, 0)),
          pl.BlockSpec((1, _N, 1), lambda b: (b, 0, 0)),
          pl.BlockSpec((_H, _H), lambda b: (0, 0)),
          pl.BlockSpec((1, _H), lambda b: (0, 0)),
          pl.BlockSpec((1, 1, _DIN), lambda b: (b, 0, 0)),
          pl.BlockSpec((_DIN, 4 * _H), lambda b: (0, 0)),
          pl.BlockSpec((1, 4 * _H), lambda b: (0, 0)),
          pl.BlockSpec((1, _N, _H), lambda b: (b, 0, 0)),
      ],
      out_specs=[
          pl.BlockSpec((1, _N, _H), lambda b: (b, 0, 0)),
          pl.BlockSpec((1, _N, _H), lambda b: (b, 0, 0)),
      ],
      out_shape=[jax.ShapeDtypeStruct((_B, _N, _H), jnp.float32),
                 jax.ShapeDtypeStruct((_B, _N, _H), jnp.float32)],
  )(parts_b, normd_col, w_g, b_g, x3, w4t, b4, c_prev)


@jax.jit
def kernel(g_batch_edge_index, x, h_prev, c_prev,
           w_i, b_i, w_f, b_f, w_o, b_o, w_c, b_c, w_g, b_g):
  src = g_batch_edge_index[0].astype(jnp.int32)
  dst = g_batch_edge_index[1].astype(jnp.int32)
  pad = jnp.full((_EPAD - _E,), _NODES, jnp.int32)
  src_flat = jnp.concatenate([src, pad])
  src_p = src_flat.reshape(_NW, _ITERS, _CHUNK)
  src_p16 = src_flat.reshape(_NS, _ITERS2, _CHUNK)
  dst_p = jnp.concatenate([dst, pad]).reshape(_NW, _ITERS, _CHUNK)
  dst_p16 = jnp.concatenate(
      [dst, jnp.zeros((_EPAD - _E,), jnp.int32)]).reshape(_NS, _ITERS2, _CHUNK)

  h_flat = h_prev.reshape(_NODES, _H)
  h_pad = jnp.concatenate(
      [h_flat, jnp.zeros((_NPAD - _NODES, _H), jnp.float32)])

  w4t = jnp.concatenate([w_i, w_f, w_o, w_c]).T
  b4 = jnp.concatenate([b_i, b_f, b_o, b_c])[None, :]

  od_p, id_p = _sc_degrees(src_p, dst_p)
  tq4, normd3 = _tc_norm(od_p.reshape(_NW, _NR, 128),
                         id_p.reshape(_NW, _NR, 128),
                         h_pad.reshape(_NR, 128, _H))
  tab = tq4.reshape(4, _NPAD, _HQ)
  normd_col = normd3.reshape(_NPAD)[:_NODES].reshape(_B, _N, 1)
  parts = _sc_aggregate(tab, src_p16, dst_p16)
  parts_b = parts.reshape(_NC, 2, _B, _N, _HQ)
  h_t, c_t = _tc_dense(parts_b, normd_col, w_g, b_g[None, :], x[:, None, :],
                       w4t, b4, c_prev)
  return (h_t, c_t)


# speedup vs baseline: 1.3955x; 1.0129x over previous
import dataclasses
import functools
import jax
import jax.numpy as jnp
from jax import lax
from jax.experimental import pallas as pl
from jax.experimental.pallas import tpu as pltpu
from jax.experimental.pallas import tpu_sc as plsc

_B, _N, _H, _DIN, _E = 4, 2500, 128, 256, 320000
_NODES = _B * _N
_NPAD = 10240
_NC, _NS = 2, 16
_NW = _NC * _NS
_CHUNK = 128
_ITERS = 80
_EPAD = _NW * _ITERS * _CHUNK
_RPS = _NPAD // _NS
_OPS = _NODES // _NS

_mesh = plsc.VectorSubcoreMesh(core_axis_name="c", subcore_axis_name="s")

_cp = pltpu.CompilerParams()
if "needs_layout_passes" in pltpu.CompilerParams.__dataclass_fields__:
  _cp = dataclasses.replace(_cp, needs_layout_passes=False)
if "use_tc_tiling_on_sc" in pltpu.CompilerParams.__dataclass_fields__:
  _cp = dataclasses.replace(_cp, use_tc_tiling_on_sc=False)

_cp_lin = pltpu.CompilerParams()
if "use_tc_tiling_on_sc" in pltpu.CompilerParams.__dataclass_fields__:
  _cp_lin = dataclasses.replace(_cp_lin, use_tc_tiling_on_sc=False)


@functools.partial(
    pl.kernel,
    out_type=[jax.ShapeDtypeStruct((_NW, _NPAD), jnp.float32),
              jax.ShapeDtypeStruct((_NW, _NPAD), jnp.float32)],
    mesh=_mesh,
    compiler_params=_cp,
    scratch_types=[pltpu.VMEM((_ITERS, _CHUNK), jnp.int32),
                   pltpu.VMEM((_ITERS, _CHUNK), jnp.int32),
                   pltpu.VMEM((_NPAD,), jnp.float32),
                   pltpu.VMEM((_NPAD,), jnp.float32)],
)
def _sc_degrees(src_h, dst_h, od_h, id_h, src_v, dst_v, od_v, id_v):
  cid = lax.axis_index("c")
  sid = lax.axis_index("s")
  wid = sid * _NC + cid

  @pl.loop(0, _NPAD, step=16)
  def _(i):
    od_v[pl.ds(i, 16)] = jnp.zeros((16,), jnp.float32)
    id_v[pl.ds(i, 16)] = jnp.zeros((16,), jnp.float32)

  pltpu.sync_copy(src_h.at[wid], src_v)
  pltpu.sync_copy(dst_h.at[wid], dst_v)

  ones = jnp.ones((16,), jnp.float32)

  @pl.loop(0, _ITERS)
  def _(i):
    @pl.loop(0, _CHUNK, step=16)
    def _(j):
      plsc.addupdate_scatter(od_v, [src_v[i, pl.ds(j, 16)]], ones)
      plsc.addupdate_scatter(id_v, [dst_v[i, pl.ds(j, 16)]], ones)

  pltpu.sync_copy(od_v, od_h.at[wid])
  pltpu.sync_copy(id_v, id_h.at[wid])


_HQ = _H // 4
_ITERS2 = _EPAD // (_NS * _CHUNK)
_TROWS = _NODES + 16
_TPS = _TROWS // _NS


@functools.partial(
    pl.kernel,
    out_type=jax.ShapeDtypeStruct((_NC, 2, _NODES, _HQ), jnp.float32),
    mesh=_mesh,
    compiler_params=_cp_lin,
    scratch_types=([pltpu.VMEM((_ITERS2, _CHUNK), jnp.int32),
                    pltpu.VMEM((_ITERS2, _CHUNK), jnp.int32)] +
                   [pltpu.VMEM((_CHUNK, _HQ), jnp.float32)] * 9 +
                   [pltpu.VMEM_SHARED((_TROWS, _HQ), jnp.float32),
                    pltpu.VMEM_SHARED((_NODES, _HQ), jnp.float32)] +
                   [pltpu.SemaphoreType.DMA] * 16),
)
def _sc_aggregate(tab_h, src_h, dst_h, out_h, src_v, dst_v,
                  r0, r1, r2, r3, r4, r5, r6, r7, zb_v, table_sh, agg_sh,
                  g0, g1, g2, g3, g4, g5, g6, g7,
                  s0, s1, s2, s3, s4, s5, s6, s7):
  cid = lax.axis_index("c")
  sid = lax.axis_index("s")

  @pl.loop(0, _CHUNK)
  def _(i):
    @pl.loop(0, _HQ, step=16)
    def _(j):
      zb_v[i, pl.ds(j, 16)] = jnp.zeros((16,), jnp.float32)

  pltpu.sync_copy(src_h.at[sid], src_v)
  pltpu.sync_copy(dst_h.at[sid], dst_v)

  obase = sid * _OPS
  tbase = sid * _TPS
  for q in (0, 1):
    @pl.loop(0, _OPS, step=125)
    def _(r):
      pltpu.sync_copy(zb_v.at[pl.ds(0, 125)], agg_sh.at[pl.ds(obase + r, 125)])

    pltpu.sync_copy(tab_h.at[cid * 2 + q, pl.ds(tbase, _TPS)],
                    table_sh.at[pl.ds(tbase, _TPS)])
    plsc.subcore_barrier()

    bufs = (r0, r1, r2, r3, r4, r5, r6, r7)
    gs = (g0, g1, g2, g3, g4, g5, g6, g7)
    ss = (s0, s1, s2, s3, s4, s5, s6, s7)
    for b in range(8):
      pltpu.async_copy(table_sh.at[src_v.at[b]], bufs[b], gs[b])

    @pl.loop(0, _ITERS2, step=8)
    def _(c):
      for b in range(8):
        cc = c + b
        pltpu.make_async_copy(table_sh.at[src_v.at[cc]], bufs[b],
                              gs[b]).wait()
        pltpu.async_copy(bufs[b], agg_sh.at[dst_v.at[cc]], ss[b], add=True)
        b4 = (b + 4) % 8

        @pl.when((cc >= 4) & (cc + 4 < _ITERS2))
        def _():
          pltpu.make_async_copy(bufs[b4], agg_sh.at[dst_v.at[cc - 4]],
                                ss[b4]).wait()
          pltpu.async_copy(table_sh.at[src_v.at[cc + 4]], bufs[b4], gs[b4])

    for b in range(8):
      pltpu.make_async_copy(bufs[b], agg_sh.at[dst_v.at[_ITERS2 - 8 + b]],
                            ss[b]).wait()

    plsc.subcore_barrier()
    pltpu.sync_copy(agg_sh.at[pl.ds(obase, _OPS)],
                    out_h.at[cid, q, pl.ds(obase, _OPS)])


_NR = _NPAD // 128


def _tc_norm_body(od_ref, id_ref, h_ref, tq_ref, normd_ref):
  od = jnp.sum(od_ref[...], axis=0)
  idg = jnp.sum(id_ref[...], axis=0)
  norm_src = jnp.where(od > 0, lax.rsqrt(jnp.maximum(od, 1.0)), 0.0)
  norm_dst = jnp.where(idg > 0, lax.rsqrt(jnp.maximum(idg, 1.0)), 0.0)
  table = h_ref[...] * norm_src[:, :, None]
  for q in range(4):
    tq_ref[q] = table[:, :, q * _HQ:(q + 1) * _HQ]
  normd_ref[...] = norm_dst


def _tc_norm(od_p, id_p, h3):
  return pl.pallas_call(
      _tc_norm_body,
      out_shape=[jax.ShapeDtypeStruct((4, _NR, 128, _HQ), jnp.float32),
                 jax.ShapeDtypeStruct((_NR, 128), jnp.float32)],
  )(od_p, id_p, h3)


def _tc_dense_body(parts_ref, normd_ref, wg_ref, bg_ref, x_ref, w4t_ref,
                   b4_ref, cprev_ref, h_ref, c_ref):
  agg = jnp.concatenate(
      [parts_ref[0, 0, 0], parts_ref[0, 1, 0],
       parts_ref[1, 0, 0], parts_ref[1, 1, 0]], axis=1)
  aggn = agg * normd_ref[0]
  hconv = jnp.dot(aggn, wg_ref[...],
                  preferred_element_type=jnp.float32) + bg_ref[...]
  xg = jnp.dot(x_ref[0], w4t_ref[...],
               preferred_element_type=jnp.float32) + b4_ref[...]
  i_t = jax.nn.sigmoid(hconv + xg[:, 0:_H])
  f_t = jax.nn.sigmoid(hconv + xg[:, _H:2 * _H])
  o_t = jax.nn.sigmoid(hconv + xg[:, 2 * _H:3 * _H])
  c_til = jnp.tanh(hconv + xg[:, 3 * _H:4 * _H])
  c_t = f_t * cprev_ref[0] + i_t * c_til
  h_ref[0] = o_t * jnp.tanh(c_t)
  c_ref[0] = c_t


def _tc_dense(parts_b, normd_col, w_g, b_g, x3, w4t, b4, c_prev):
  return pl.pallas_call(
      _tc_dense_body,
      grid=(_B,),
      in_specs=[
          pl.BlockSpec((_NC, 2, 1, _N, _HQ), lambda b: (0, 0, b, 0, 0)),
          pl.BlockSpec((1, _N, 1), lambda b: (b, 0, 0)),
          pl.BlockSpec((_H, _H), lambda b: (0, 0)),
          pl.BlockSpec((1, _H), lambda b: (0, 0)),
          pl.BlockSpec((1, 1, _DIN), lambda b: (b, 0, 0)),
          pl.BlockSpec((_DIN, 4 * _H), lambda b: (0, 0)),
          pl.BlockSpec((1, 4 * _H), lambda b: (0, 0)),
          pl.BlockSpec((1, _N, _H), lambda b: (b, 0, 0)),
      ],
      out_specs=[
          pl.BlockSpec((1, _N, _H), lambda b: (b, 0, 0)),
          pl.BlockSpec((1, _N, _H), lambda b: (b, 0, 0)),
      ],
      out_shape=[jax.ShapeDtypeStruct((_B, _N, _H), jnp.float32),
                 jax.ShapeDtypeStruct((_B, _N, _H), jnp.float32)],
  )(parts_b, normd_col, w_g, b_g, x3, w4t, b4, c_prev)


@jax.jit
def kernel(g_batch_edge_index, x, h_prev, c_prev,
           w_i, b_i, w_f, b_f, w_o, b_o, w_c, b_c, w_g, b_g):
  src = g_batch_edge_index[0].astype(jnp.int32)
  dst = g_batch_edge_index[1].astype(jnp.int32)
  pad = jnp.full((_EPAD - _E,), _NODES, jnp.int32)
  src_flat = jnp.concatenate([src, pad])
  src_p = src_flat.reshape(_NW, _ITERS, _CHUNK)
  src_p16 = src_flat.reshape(_NS, _ITERS2, _CHUNK)
  dst_p = jnp.concatenate([dst, pad]).reshape(_NW, _ITERS, _CHUNK)
  dst_p16 = jnp.concatenate(
      [dst, jnp.zeros((_EPAD - _E,), jnp.int32)]).reshape(_NS, _ITERS2, _CHUNK)

  h_flat = h_prev.reshape(_NODES, _H)
  h_pad = jnp.concatenate(
      [h_flat, jnp.zeros((_NPAD - _NODES, _H), jnp.float32)])

  w4t = jnp.concatenate([w_i, w_f, w_o, w_c]).T
  b4 = jnp.concatenate([b_i, b_f, b_o, b_c])[None, :]

  od_p, id_p = _sc_degrees(src_p, dst_p)
  tq4, normd3 = _tc_norm(od_p.reshape(_NW, _NR, 128),
                         id_p.reshape(_NW, _NR, 128),
                         h_pad.reshape(_NR, 128, _H))
  tab = tq4.reshape(4, _NPAD, _HQ)
  normd_col = normd3.reshape(_NPAD)[:_NODES].reshape(_B, _N, 1)
  parts = _sc_aggregate(tab, src_p16, dst_p16)
  parts_b = parts.reshape(_NC, 2, _B, _N, _HQ)
  h_t, c_t = _tc_dense(parts_b, normd_col, w_g, b_g[None, :], x[:, None, :],
                       w4t, b4, c_prev)
  return (h_t, c_t)
